# baseline (speedup 1.0000x reference)
import functools

import jax
import jax.numpy as jnp
from jax import lax
from jax.experimental import pallas as pl
from jax.experimental.pallas import tpu as pltpu
from jax.experimental.pallas import tpu_sc as plsc

BATCH = 16384
EMBED_DIM = 32
NROWS = 1000000
PACK = 4
GSIZE = 1 << 18
PROWS = GSIZE
PWIDTH = EMBED_DIM * PACK

_INFO = plsc.get_sparse_core_info()
_NC = _INFO.num_cores
_NS = _INFO.num_subcores
_NW = _NC * _NS
_BPW = BATCH // _NW


_UBLK = 16384

_CONTRACT00 = (((0,), (0,)), ((), ()))


def _repack_body(t0_ref, t1_ref, t2_ref, t3_ref, eye_ref, o_ref):
    x = jnp.concatenate(
        [t0_ref[...], t1_ref[...], t2_ref[...], t3_ref[...]], axis=0)
    o_ref[...] = jax.lax.dot_general(x, eye_ref[...], _CONTRACT00,
                                     preferred_element_type=jnp.float32)


def _tc_repack(tab_t, eye):
    grid = (GSIZE // _UBLK,)
    gstride = GSIZE // _UBLK
    last_blk = NROWS // _UBLK

    def in_map(g):
        return lambda i: (0, jnp.minimum(g * gstride + i, last_blk))

    return pl.pallas_call(
        _repack_body,
        grid=grid,
        in_specs=(
            [pl.BlockSpec((EMBED_DIM, _UBLK), in_map(g)) for g in range(PACK)]
            + [pl.BlockSpec((PWIDTH, PWIDTH), lambda i: (0, 0))]
        ),
        out_specs=pl.BlockSpec((_UBLK, PWIDTH), lambda i: (i, 0)),
        out_shape=jax.ShapeDtypeStruct((PROWS, PWIDTH), jnp.float32),
    )(tab_t, tab_t, tab_t, tab_t, eye)


@functools.partial(
    pl.kernel,
    mesh=plsc.VectorSubcoreMesh(core_axis_name="c", subcore_axis_name="s"),
    out_type=jax.ShapeDtypeStruct((EMBED_DIM, BATCH), jnp.float32),
    scratch_types=[
        pltpu.VMEM((_BPW,), jnp.int32),
        [pltpu.VMEM((_BPW // 4,), jnp.int32) for _ in range(4)],
        [pltpu.VMEM((_BPW // 4, PWIDTH), jnp.float32) for _ in range(4)],
        pltpu.VMEM((EMBED_DIM, _BPW), jnp.float32),
        [pltpu.SemaphoreType.DMA for _ in range(4)],
    ],
    compiler_params=pltpu.CompilerParams(needs_layout_passes=False),
)
def _sc_gather(ids_hbm, ptab_hbm, out_hbm, ids_v, row_v, rows_v, cols_v, sems):
    wid = lax.axis_index("s") * _NC + lax.axis_index("c")
    base = wid * _BPW
    chunk = _BPW // 4
    pltpu.sync_copy(ids_hbm.at[pl.ds(base, _BPW)], ids_v)
    copies = []
    for c4 in range(4):
        for j0 in range(0, chunk, 16):
            row_v[c4][pl.ds(j0, 16)] = (
                ids_v[pl.ds(c4 * chunk + j0, 16)] & (GSIZE - 1))
        copies.append(
            pltpu.async_copy(ptab_hbm.at[row_v[c4]], rows_v[c4], sems[c4]))
    iota16 = lax.iota(jnp.int32, 16)

    for c4 in range(4):
        copies[c4].wait()

        def extract(g, carry, c4=c4):
            j0 = g * 16
            grp = ids_v[pl.ds(c4 * chunk + j0, 16)] >> 18
            col0 = grp * EMBED_DIM
            ridx = iota16 + j0
            for c in range(EMBED_DIM):
                cols_v[c, pl.ds(c4 * chunk + j0, 16)] = plsc.load_gather(
                    rows_v[c4], [ridx, col0 + c])
            return carry

        lax.fori_loop(0, chunk // 16, extract, 0)
    pltpu.sync_copy(cols_v, out_hbm.at[:, pl.ds(base, _BPW)])


_BB = 4096


def _mlp_body(xu_ref, xi_ref, w1u_ref, w1i_ref, b1_ref, w2_ref, b2t_ref,
              o_ref):
    h = (
        jax.lax.dot_general(xu_ref[...], w1u_ref[...], _CONTRACT00,
                            preferred_element_type=jnp.float32)
        + jax.lax.dot_general(xi_ref[...], w1i_ref[...], _CONTRACT00,
                              preferred_element_type=jnp.float32)
        + b1_ref[...]
    )
    h = jnp.maximum(h, 0.0)
    o_ref[...] = (
        jax.lax.dot_general(w2_ref[...], h, (((0,), (1,)), ((), ())),
                            preferred_element_type=jnp.float32)
        + b2t_ref[...]
    )


def _tc_mlp_t(xu_t, xi_t, w1u, w1i, b1, w2, b2t):
    grid = (BATCH // _BB,)
    return pl.pallas_call(
        _mlp_body,
        grid=grid,
        in_specs=[
            pl.BlockSpec((EMBED_DIM, _BB), lambda i: (0, i)),
            pl.BlockSpec((EMBED_DIM, _BB), lambda i: (0, i)),
            pl.BlockSpec((EMBED_DIM, 64), lambda i: (0, 0)),
            pl.BlockSpec((EMBED_DIM, 64), lambda i: (0, 0)),
            pl.BlockSpec((1, 64), lambda i: (0, 0)),
            pl.BlockSpec((64, 11), lambda i: (0, 0)),
            pl.BlockSpec((11, 1), lambda i: (0, 0)),
        ],
        out_specs=pl.BlockSpec((11, _BB), lambda i: (0, i)),
        out_shape=jax.ShapeDtypeStruct((11, BATCH), jnp.float32),
    )(xu_t, xi_t, w1u, w1i, b1, w2, b2t)


def kernel(user_ids, item_ids, user_table, item_table, W1, b1, W2, b2):
    uid = user_ids.astype(jnp.int32)
    iid = item_ids.astype(jnp.int32)
    eye = jnp.eye(PWIDTH, dtype=jnp.float32)
    uptab = _tc_repack(user_table.T, eye)
    urows_t = _sc_gather(uid, uptab)
    iptab = _tc_repack(item_table.T, eye)
    irows_t = _sc_gather(iid, iptab)
    out_t = _tc_mlp_t(
        urows_t, irows_t,
        W1[:EMBED_DIM], W1[EMBED_DIM:],
        b1.reshape(1, 64), W2, b2.reshape(11, 1),
    )
    return out_t.T


# speedup vs baseline: 1.0070x; 1.0070x over previous
import functools

import jax
import jax.numpy as jnp
from jax import lax
from jax.experimental import pallas as pl
from jax.experimental.pallas import tpu as pltpu
from jax.experimental.pallas import tpu_sc as plsc

BATCH = 16384
EMBED_DIM = 32
NROWS = 1000000
PACK = 4
GSIZE = 1 << 18
PROWS = GSIZE
PWIDTH = EMBED_DIM * PACK

_INFO = plsc.get_sparse_core_info()
_NC = _INFO.num_cores
_NS = _INFO.num_subcores
_NW = _NC * _NS
_BPW = BATCH // _NW


_UBLK = 16384

_CONTRACT00 = (((0,), (0,)), ((), ()))


def _repack_body(t0_ref, t1_ref, t2_ref, t3_ref, eye_ref, o_ref):
    x = jnp.concatenate(
        [t0_ref[...], t1_ref[...], t2_ref[...], t3_ref[...]], axis=0)
    o_ref[...] = jax.lax.dot_general(x, eye_ref[...], _CONTRACT00,
                                     preferred_element_type=jnp.float32)


def _tc_repack(tab_t, eye):
    grid = (GSIZE // _UBLK,)
    gstride = GSIZE // _UBLK
    last_blk = NROWS // _UBLK

    def in_map(g):
        return lambda i: (0, jnp.minimum(g * gstride + i, last_blk))

    return pl.pallas_call(
        _repack_body,
        grid=grid,
        in_specs=(
            [pl.BlockSpec((EMBED_DIM, _UBLK), in_map(g)) for g in range(PACK)]
            + [pl.BlockSpec((PWIDTH, PWIDTH), lambda i: (0, 0))]
        ),
        out_specs=pl.BlockSpec((_UBLK, PWIDTH), lambda i: (i, 0)),
        out_shape=jax.ShapeDtypeStruct((PROWS, PWIDTH), jnp.float32),
    )(tab_t, tab_t, tab_t, tab_t, eye)


@functools.partial(
    pl.kernel,
    mesh=plsc.VectorSubcoreMesh(core_axis_name="c", subcore_axis_name="s"),
    out_type=jax.ShapeDtypeStruct((EMBED_DIM, BATCH), jnp.float32),
    scratch_types=[
        pltpu.VMEM((_BPW,), jnp.int32),
        pltpu.VMEM((_BPW,), jnp.int32),
        pltpu.VMEM((_BPW, PWIDTH), jnp.float32),
        pltpu.VMEM((EMBED_DIM, _BPW), jnp.float32),
        pltpu.SemaphoreType.DMA,
    ],
    compiler_params=pltpu.CompilerParams(needs_layout_passes=False),
)
def _sc_gather(ids_hbm, ptab_hbm, out_hbm, ids_v, row_v, rows_v, cols_v, sem):
    wid = lax.axis_index("s") * _NC + lax.axis_index("c")
    base = wid * _BPW
    pltpu.sync_copy(ids_hbm.at[pl.ds(base, _BPW)], ids_v)
    for j0 in range(0, _BPW, 16):
        row_v[pl.ds(j0, 16)] = ids_v[pl.ds(j0, 16)] & (GSIZE - 1)
    copy = pltpu.async_copy(ptab_hbm.at[row_v], rows_v, sem)
    iota16 = lax.iota(jnp.int32, 16)
    copy.wait()

    def extract(g, carry):
        j0 = g * 16
        grp = ids_v[pl.ds(j0, 16)] >> 18
        col0 = grp * EMBED_DIM
        ridx = iota16 + j0
        for c in range(EMBED_DIM):
            cols_v[c, pl.ds(j0, 16)] = plsc.load_gather(
                rows_v, [ridx, col0 + c])
        return carry

    lax.fori_loop(0, _BPW // 16, extract, 0)
    pltpu.sync_copy(cols_v, out_hbm.at[:, pl.ds(base, _BPW)])


_BB = 4096


def _mlp_body(xu_ref, xi_ref, w1u_ref, w1i_ref, b1_ref, w2_ref, b2t_ref,
              o_ref):
    h = (
        jax.lax.dot_general(xu_ref[...], w1u_ref[...], _CONTRACT00,
                            preferred_element_type=jnp.float32)
        + jax.lax.dot_general(xi_ref[...], w1i_ref[...], _CONTRACT00,
                              preferred_element_type=jnp.float32)
        + b1_ref[...]
    )
    h = jnp.maximum(h, 0.0)
    o_ref[...] = (
        jax.lax.dot_general(w2_ref[...], h, (((0,), (1,)), ((), ())),
                            preferred_element_type=jnp.float32)
        + b2t_ref[...]
    )


def _tc_mlp_t(xu_t, xi_t, w1u, w1i, b1, w2, b2t):
    grid = (BATCH // _BB,)
    return pl.pallas_call(
        _mlp_body,
        grid=grid,
        in_specs=[
            pl.BlockSpec((EMBED_DIM, _BB), lambda i: (0, i)),
            pl.BlockSpec((EMBED_DIM, _BB), lambda i: (0, i)),
            pl.BlockSpec((EMBED_DIM, 64), lambda i: (0, 0)),
            pl.BlockSpec((EMBED_DIM, 64), lambda i: (0, 0)),
            pl.BlockSpec((1, 64), lambda i: (0, 0)),
            pl.BlockSpec((64, 11), lambda i: (0, 0)),
            pl.BlockSpec((11, 1), lambda i: (0, 0)),
        ],
        out_specs=pl.BlockSpec((11, _BB), lambda i: (0, i)),
        out_shape=jax.ShapeDtypeStruct((11, BATCH), jnp.float32),
    )(xu_t, xi_t, w1u, w1i, b1, w2, b2t)


def kernel(user_ids, item_ids, user_table, item_table, W1, b1, W2, b2):
    uid = user_ids.astype(jnp.int32)
    iid = item_ids.astype(jnp.int32)
    eye = jnp.eye(PWIDTH, dtype=jnp.float32)
    uptab = _tc_repack(user_table.T, eye)
    urows_t = _sc_gather(uid, uptab)
    iptab = _tc_repack(item_table.T, eye)
    irows_t = _sc_gather(iid, iptab)
    out_t = _tc_mlp_t(
        urows_t, irows_t,
        W1[:EMBED_DIM], W1[EMBED_DIM:],
        b1.reshape(1, 64), W2, b2.reshape(11, 1),
    )
    return out_t.T
